# Initial kernel scaffold; baseline (speedup 1.0000x reference)
import jax
import jax.numpy as jnp
from jax.experimental import pallas as pl


def kernel(node_f, node_x, edge_index, edge_attr, graph_ids, params):
    raise NotImplementedError("write your pallas kernel here")


# speedup vs baseline: 5.8673x; 5.8673x over previous
import functools

import jax
import jax.numpy as jnp
import numpy as np
from jax import lax
from jax.experimental import pallas as pl
from jax.experimental.pallas import tpu as pltpu
from jax.experimental.pallas import tpu_sc as plsc

N = 10000
E = 320000
D = 128
H = 4
DH = 32
VC = 8
G = 512

_NC, _NS = 2, 16
_NW = _NC * _NS
_CH = 80
_PER = E // _NW
_NCHUNK = _PER // _CH
_IDXROWS = E // _CH
_IPW = _IDXROWS // _NW
_NACC = 10240
_RPT = _NACC // _NS
_DPT = 400
_BN = 1000
_BE = 2000
_PC = 128

_INV_SQRT_DH = float(1.0 / np.sqrt(DH))

_SEL = np.repeat(np.eye(H, dtype=np.float32), DH, axis=0)
_SELT = np.ascontiguousarray(_SEL.T)
_A24 = np.kron(np.eye(VC, dtype=np.float32), np.ones((1, 3), np.float32))
_A24T = np.ascontiguousarray(_A24.T)
_B3 = np.kron(np.ones((1, VC), np.float32), np.eye(3, dtype=np.float32))
_B16 = np.zeros((16, 24), np.float32)
_B16[6:9] = _B3
_B3_16 = np.zeros((16, 24), np.float32)
_B3_16[0:3] = _B3


def _lnk(x):
    m = jnp.mean(x, axis=-1, keepdims=True)
    xc = x - m
    v = jnp.mean(xc * xc, axis=-1, keepdims=True)
    return xc / jnp.sqrt(v + 1e-5)


def _mlp2k(x, w1, b1, w2, b2):
    h = jax.nn.relu(_lnk(x @ w1 + b1))
    return h @ w2 + b2


def _sc_mesh():
    return plsc.VectorSubcoreMesh(core_axis_name="c", subcore_axis_name="s")


@functools.partial(
    pl.kernel,
    mesh=_sc_mesh(),
    out_type=(jax.ShapeDtypeStruct((E, D), jnp.float32),
              jax.ShapeDtypeStruct((E, D), jnp.float32)),
    scratch_types=[
        pltpu.VMEM((_IPW, _CH), jnp.int32),
        pltpu.VMEM((_IPW, _CH), jnp.int32),
        pltpu.VMEM((_CH, D), jnp.float32),
        pltpu.VMEM((_CH, D), jnp.float32),
        pltpu.SemaphoreType.DMA,
    ],
)
def _sc_gather_pair(t1, i1, t2, i2, o1, o2, iv1, iv2, r1, r2, sem):
    wid = lax.axis_index("s") * _NC + lax.axis_index("c")
    base = wid * _PER
    pltpu.sync_copy(i1.at[wid], iv1)
    pltpu.sync_copy(i2.at[wid], iv2)

    def body(j, carry):
        off = base + j * _CH
        pltpu.async_copy(t1.at[iv1.at[j]], r1, sem).wait()
        pltpu.sync_copy(r1, o1.at[pl.ds(off, _CH), :])
        pltpu.async_copy(t2.at[iv2.at[j]], r2, sem).wait()
        pltpu.sync_copy(r2, o2.at[pl.ds(off, _CH), :])
        return carry

    lax.fori_loop(0, _NCHUNK, body, 0)


@functools.partial(
    pl.kernel,
    mesh=_sc_mesh(),
    out_type=jax.ShapeDtypeStruct((_NC, N, _PC), jnp.float32),
    scratch_types=[
        pltpu.VMEM((_IPW, _CH), jnp.int32),
        pltpu.VMEM((_CH, _PC), jnp.float32),
        pltpu.VMEM_SHARED((_NACC, _PC), jnp.float32),
        pltpu.SemaphoreType.DMA,
    ],
)
def _sc_scatter_add(pay, idx, zer, out, iv, pv, acc, sem):
    cid = lax.axis_index("c")
    sid = lax.axis_index("s")
    wid = sid * _NC + cid
    base = wid * _PER

    pltpu.sync_copy(zer, acc.at[pl.ds(sid * _RPT, _RPT), :])
    plsc.subcore_barrier()

    @pl.when(sid == 0)
    def _scatter_all():
        def wbody(s, carry):
            w = s * _NC + cid
            pltpu.sync_copy(idx.at[w], iv)

            def body(j, carry2):
                off = w * _PER + j * _CH
                pltpu.sync_copy(pay.at[pl.ds(off, _CH), :], pv)
                pltpu.sync_copy(pv, acc.at[iv.at[j]], add=True)
                return carry2

            lax.fori_loop(0, _NCHUNK, body, 0)
            return carry

        lax.fori_loop(0, _NS, wbody, 0)

    plsc.subcore_barrier()

    @pl.when(sid < _NS - 1)
    def _dump_full():
        pltpu.sync_copy(acc.at[pl.ds(sid * _RPT, _RPT), :],
                        out.at[cid, pl.ds(sid * _RPT, _RPT), :])

    @pl.when(sid == _NS - 1)
    def _dump_tail():
        pltpu.sync_copy(acc.at[pl.ds(sid * _RPT, _DPT), :],
                        out.at[cid, pl.ds(sid * _RPT, _DPT), :])


def _embed_kernel(f_ref, x_ref, w8, br, b316, ver, qw1, qb1, qw2, qb2,
                  s_out, v_out, q_out):
    f = f_ref[...]
    s0 = f @ w8[...] + br[...]
    v0 = (x_ref[...] @ b316[...]) * ver[...]
    q0 = _mlp2k(s0, qw1[...], qb1[...], qw2[...], qb2[...])
    s_out[...] = s0
    v_out[...] = v0
    q_out[...] = q0


def _aux_kernel(xs_ref, xd_ref, ea_ref, aux_out):
    xs = xs_ref[...]
    xd = xd_ref[...]
    rel = xs[:, 0:3] - xd[:, 0:3]
    d2 = jnp.sum(rel * rel, axis=-1, keepdims=True) + 1e-12
    dist = jnp.sqrt(d2)
    ru = rel / (dist + 1e-8)
    ea = ea_ref[...][:, 0:5]
    zeros = jnp.zeros((ea.shape[0], 7), jnp.float32)
    aux_out[...] = jnp.concatenate([ea, dist, ru, zeros], axis=-1)


def _edge_kernel(s_ref, q_ref, aux_ref,
                 wk1a, wk1b, kb1, kw2, kb2,
                 wv1a, wv1b, vb1, vw2, vb2,
                 gw, gb, sel, selt, a24, b16, out_ref, out2_ref):
    s = s_ref[...]
    qd = q_ref[...]
    aux = aux_ref[...]
    hk = jax.nn.relu(_lnk(s @ wk1a[...] + aux @ wk1b[...] + kb1[...]))
    k = hk @ kw2[...] + kb2[...]
    hv = jax.nn.relu(_lnk(s @ wv1a[...] + aux @ wv1b[...] + vb1[...]))
    v = hv @ vw2[...] + vb2[...]
    logits = ((qd * k) @ sel[...]) * _INV_SQRT_DH
    ex = jnp.exp(logits)
    wv = v * (ex @ selt[...])
    gate = jnp.tanh(v @ gw[...] + gb[...])
    vm = (gate @ a24[...]) * (aux @ b16[...])
    zeros = jnp.zeros((s.shape[0], 100), jnp.float32)
    out_ref[...] = wv
    out2_ref[...] = jnp.concatenate([ex, vm, zeros], axis=-1)


def _node_kernel(part_ref, part2_ref, s_ref, v_ref,
                 selt, ow1, ob1, ow2, ob2, vsr, a24, a24t,
                 qw1, qb1, qw2, qb2,
                 s_out, v_out, q_out):
    pr = part_ref[...]
    numer = pr[0] + pr[1]
    pr2 = part2_ref[...]
    sall2 = pr2[0] + pr2[1]
    den4 = sall2[:, 0:4]
    vagg = sall2[:, 4:28]
    agg = numer / (den4 @ selt[...] + 1e-9)
    upd = _mlp2k(agg, ow1[...], ob1[...], ow2[...], ob2[...])
    s1 = _lnk(s_ref[...] + upd)
    vr = v_ref[...] + vagg
    v2 = (vr * vr) @ a24t[...]
    vnorm = jnp.sqrt(v2 + 1e-12)
    fac = vsr[...] / (1.0 + vnorm)
    v1 = vr * (fac @ a24[...])
    qn = _mlp2k(s1, qw1[...], qb1[...], qw2[...], qb2[...])
    s_out[...] = s1
    v_out[...] = v1
    q_out[...] = qn


def _feat_kernel(s_ref, v_ref, a24t, w1a, w1b, b1, w2, b2, f_out):
    sc = s_ref[...]
    vr = v_ref[...]
    vinv = jnp.sqrt((vr * vr) @ a24t[...] + 1e-12)
    h = jax.nn.relu(_lnk(sc @ w1a[...] + vinv @ w1b[...] + b1[...]))
    f_out[...] = h @ w2[...] + b2[...] + sc


def _pool_kernel(ids_ref, f_ref, gw1, gb1, gw2, gb2, out_ref):
    step = pl.program_id(0)

    @pl.when(step == 0)
    def _init():
        out_ref[...] = jnp.full((G, D), -1e30, jnp.float32)

    def body(r, carry):
        g = ids_ref[0, 0, r]
        row = f_ref[pl.ds(r, 1), :]
        out_ref[pl.ds(g, 1), :] = jnp.maximum(out_ref[pl.ds(g, 1), :], row)
        return carry

    lax.fori_loop(0, _BN, body, 0)

    @pl.when(step == pl.num_programs(0) - 1)
    def _head():
        pooled = out_ref[...]
        pooled = jnp.where(pooled == -1e30, 0.0, pooled)
        h = jax.nn.relu(_lnk(pooled @ gw1[...] + gb1[...]))
        out_ref[...] = h @ gw2[...] + gb2[...]


def _full(shape):
    nd = len(shape)
    return pl.BlockSpec(shape, lambda i, _nd=nd: (0,) * _nd)


def _rows(block, ncols):
    return pl.BlockSpec((block, ncols), lambda i: (i, 0))


def _tc_embed(f8, x16, w8, br, b316, ver, qw1, qb1, qw2, qb2):
    return pl.pallas_call(
        _embed_kernel,
        grid=(N // _BN,),
        in_specs=[_rows(_BN, 8), _rows(_BN, 16),
                  _full((8, D)), _full((1, D)), _full((16, 24)), _full((1, 24)),
                  _full((D, D)), _full((1, D)), _full((D, D)), _full((1, D))],
        out_specs=[_rows(_BN, D), _rows(_BN, 24), _rows(_BN, D)],
        out_shape=[jax.ShapeDtypeStruct((N, D), jnp.float32),
                   jax.ShapeDtypeStruct((N, 24), jnp.float32),
                   jax.ShapeDtypeStruct((N, D), jnp.float32)],
    )(f8, x16, w8, br, b316, ver, qw1, qb1, qw2, qb2)


def _tc_aux(xs16, xd16, ea8):
    return pl.pallas_call(
        _aux_kernel,
        grid=(E // _BE,),
        in_specs=[_rows(_BE, D), _rows(_BE, D), _rows(_BE, 8)],
        out_specs=_rows(_BE, 16),
        out_shape=jax.ShapeDtypeStruct((E, 16), jnp.float32),
    )(xs16, xd16, ea8)


def _tc_edge(ssrc, qdst, aux, wk1a, wk1b, kb1, kw2, kb2,
             wv1a, wv1b, vb1, vw2, vb2, gw, gb, sel, selt, a24, b16):
    return pl.pallas_call(
        _edge_kernel,
        grid=(E // _BE,),
        in_specs=[_rows(_BE, D), _rows(_BE, D), _rows(_BE, 16),
                  _full((D, D)), _full((16, D)), _full((1, D)),
                  _full((D, D)), _full((1, D)),
                  _full((D, D)), _full((16, D)), _full((1, D)),
                  _full((D, D)), _full((1, D)),
                  _full((D, VC)), _full((1, VC)),
                  _full((D, H)), _full((H, D)),
                  _full((VC, 24)), _full((16, 24))],
        out_specs=[_rows(_BE, _PC), _rows(_BE, _PC)],
        out_shape=[jax.ShapeDtypeStruct((E, _PC), jnp.float32),
                   jax.ShapeDtypeStruct((E, _PC), jnp.float32)],
    )(ssrc, qdst, aux, wk1a, wk1b, kb1, kw2, kb2,
      wv1a, wv1b, vb1, vw2, vb2, gw, gb, sel, selt, a24, b16)


def _tc_node(part, part2, scalar, vec, selt, ow1, ob1, ow2, ob2, vsr, a24, a24t,
             qw1, qb1, qw2, qb2):
    return pl.pallas_call(
        _node_kernel,
        grid=(N // _BN,),
        in_specs=[pl.BlockSpec((_NC, _BN, _PC), lambda i: (0, i, 0)),
                  pl.BlockSpec((_NC, _BN, _PC), lambda i: (0, i, 0)),
                  _rows(_BN, D), _rows(_BN, 24),
                  _full((H, D)),
                  _full((D, D)), _full((1, D)), _full((D, D)), _full((1, D)),
                  _full((1, VC)), _full((VC, 24)), _full((24, VC)),
                  _full((D, D)), _full((1, D)), _full((D, D)), _full((1, D))],
        out_specs=[_rows(_BN, D), _rows(_BN, 24), _rows(_BN, D)],
        out_shape=[jax.ShapeDtypeStruct((N, D), jnp.float32),
                   jax.ShapeDtypeStruct((N, 24), jnp.float32),
                   jax.ShapeDtypeStruct((N, D), jnp.float32)],
    )(part, part2, scalar, vec, selt, ow1, ob1, ow2, ob2, vsr, a24, a24t,
      qw1, qb1, qw2, qb2)


def _tc_feat(scalar, vec, a24t, w1a, w1b, b1, w2, b2):
    return pl.pallas_call(
        _feat_kernel,
        grid=(N // _BN,),
        in_specs=[_rows(_BN, D), _rows(_BN, 24), _full((24, VC)),
                  _full((D, D)), _full((VC, D)), _full((1, D)),
                  _full((D, D)), _full((1, D))],
        out_specs=_rows(_BN, D),
        out_shape=jax.ShapeDtypeStruct((N, D), jnp.float32),
    )(scalar, vec, a24t, w1a, w1b, b1, w2, b2)


def _tc_pool(ids3, feat, gw1, gb1, gw2, gb2):
    return pl.pallas_call(
        _pool_kernel,
        grid=(N // _BN,),
        in_specs=[pl.BlockSpec((1, 1, _BN), lambda i: (i, 0, 0),
                               memory_space=pltpu.SMEM),
                  _rows(_BN, D),
                  _full((D, D)), _full((1, D)), _full((D, D)), _full((1, D))],
        out_specs=_full((G, D)),
        out_shape=jax.ShapeDtypeStruct((G, D), jnp.float32),
    )(ids3, feat, gw1, gb1, gw2, gb2)


def kernel(node_f, node_x, edge_index, edge_attr, graph_ids, params):
    p = params
    src2 = edge_index[0].reshape(_NW, _IPW, _CH).astype(jnp.int32)
    dst2 = edge_index[1].reshape(_NW, _IPW, _CH).astype(jnp.int32)

    f8 = jnp.pad(node_f[:, :, 0], ((0, 0), (0, 2)))
    x16 = jnp.pad(node_x, ((0, 0), (0, 13)))
    ea8 = jnp.pad(edge_attr, ((0, 0), (0, 3)))
    zer = jnp.zeros((_RPT, _PC), jnp.float32)

    sel = jnp.asarray(_SEL)
    selt = jnp.asarray(_SELT)
    a24 = jnp.asarray(_A24)
    a24t = jnp.asarray(_A24T)
    b16 = jnp.asarray(_B16)
    b316 = jnp.asarray(_B3_16)

    embw = p['embed_W']
    embw = embw.at[5].set(embw[5] / 9.0)
    w8 = jnp.concatenate([embw, jnp.zeros((2, D), jnp.float32)], axis=0)
    br = p['embed_b'][None, :]
    ver = jnp.repeat(p['vec_embed'][0], 3)[None, :]

    lps = [p['l%d' % l] for l in range(4)]
    lp0 = lps[0]
    scalar, vec, q = _tc_embed(
        f8, x16, w8, br, b316, ver,
        lp0['qW1'], lp0['qb1'][None, :], lp0['qW2'], lp0['qb2'][None, :])

    x128 = jnp.pad(node_x, ((0, 0), (0, D - 3)))
    xs, xd = _sc_gather_pair(x128, src2, x128, dst2)
    aux = _tc_aux(xs, xd, ea8)

    for l in range(4):
        lp = lps[l]
        wk1b = jnp.concatenate(
            [lp['kW1'][D:D + 6], jnp.zeros((10, D), jnp.float32)], axis=0)
        wv1b = jnp.concatenate(
            [lp['vW1'][D:D + 6], jnp.zeros((10, D), jnp.float32)], axis=0)

        ssrc, qdst = _sc_gather_pair(scalar, src2, q, dst2)
        pay, pay2 = _tc_edge(
            ssrc, qdst, aux,
            lp['kW1'][:D], wk1b, lp['kb1'][None, :], lp['kW2'], lp['kb2'][None, :],
            lp['vW1'][:D], wv1b, lp['vb1'][None, :], lp['vW2'], lp['vb2'][None, :],
            lp['gW'], lp['gb'][None, :], sel, selt, a24, b16)
        part = _sc_scatter_add(pay, dst2, zer)
        part2 = _sc_scatter_add(pay2, dst2, zer)

        lpn = lps[(l + 1) % 4]
        scalar, vec, q = _tc_node(
            part, part2, scalar, vec, selt,
            lp['oW1'], lp['ob1'][None, :], lp['oW2'], lp['ob2'][None, :],
            lp['vscale'][None, :], a24, a24t,
            lpn['qW1'], lpn['qb1'][None, :], lpn['qW2'], lpn['qb2'][None, :])

    nmw1b = p['nmW1'][D:D + VC]
    feat = _tc_feat(scalar, vec, a24t,
                    p['nmW1'][:D], nmw1b, p['nmb1'][None, :],
                    p['nmW2'], p['nmb2'][None, :])

    gw2 = jnp.zeros((D, D), jnp.float32).at[:, 0:1].set(p['gmW2'])
    gb2 = jnp.zeros((1, D), jnp.float32).at[0, 0].set(p['gmb2'][0])
    ids3 = graph_ids.reshape(N // _BN, 1, _BN).astype(jnp.int32)
    out = _tc_pool(ids3, feat, p['gmW1'], p['gmb1'][None, :], gw2, gb2)
    return out[:, 0:1]


# speedup vs baseline: 7.7099x; 1.3140x over previous
import functools

import jax
import jax.numpy as jnp
import numpy as np
from jax import lax
from jax.experimental import pallas as pl
from jax.experimental.pallas import tpu as pltpu
from jax.experimental.pallas import tpu_sc as plsc

N = 10000
E = 320000
D = 128
H = 4
DH = 32
VC = 8
G = 512

_NC, _NS = 2, 16
_NW = _NC * _NS
_CH = 80
_PER = E // _NW
_NCHUNK = _PER // _CH
_IDXROWS = E // _CH
_IPW = _IDXROWS // _NW
_NACC = 10240
_RPT = _NACC // _NS
_DPT = 400
_BN = 1000
_K = 4
_KS = 2
_BE = 2000
_PC = 128

_INV_SQRT_DH = float(1.0 / np.sqrt(DH))

_SEL = np.repeat(np.eye(H, dtype=np.float32), DH, axis=0)
_SELT = np.ascontiguousarray(_SEL.T)
_A24 = np.kron(np.eye(VC, dtype=np.float32), np.ones((1, 3), np.float32))
_A24T = np.ascontiguousarray(_A24.T)
_B3 = np.kron(np.ones((1, VC), np.float32), np.eye(3, dtype=np.float32))
_B16 = np.zeros((16, 24), np.float32)
_B16[6:9] = _B3
_B3_16 = np.zeros((16, 24), np.float32)
_B3_16[0:3] = _B3


def _lnk(x):
    m = jnp.mean(x, axis=-1, keepdims=True)
    xc = x - m
    v = jnp.mean(xc * xc, axis=-1, keepdims=True)
    return xc / jnp.sqrt(v + 1e-5)


def _mlp2k(x, w1, b1, w2, b2):
    h = jax.nn.relu(_lnk(x @ w1 + b1))
    return h @ w2 + b2


def _sc_mesh():
    return plsc.VectorSubcoreMesh(core_axis_name="c", subcore_axis_name="s")


@functools.partial(
    pl.kernel,
    mesh=_sc_mesh(),
    out_type=(jax.ShapeDtypeStruct((E, D), jnp.float32),
              jax.ShapeDtypeStruct((E, D), jnp.float32)),
    scratch_types=(
        [pltpu.VMEM((_IPW, _CH), jnp.int32),
         pltpu.VMEM((_IPW, _CH), jnp.int32)]
        + [pltpu.VMEM((_CH, D), jnp.float32)] * (2 * _K)
        + [pltpu.SemaphoreType.DMA] * (2 * _K)
    ),
)
def _sc_gather_pair(t1, i1, t2, i2, o1, o2, iv1, iv2, *bufs):
    r1 = bufs[0:_K]
    r2 = bufs[_K:2 * _K]
    s1 = bufs[2 * _K:3 * _K]
    s2 = bufs[3 * _K:4 * _K]
    wid = lax.axis_index("s") * _NC + lax.axis_index("c")
    base = wid * _PER
    pltpu.sync_copy(i1.at[wid], iv1)
    pltpu.sync_copy(i2.at[wid], iv2)

    def body(k, carry):
        hs = []
        for b in range(_K):
            j = k * _K + b
            hs.append((pltpu.async_copy(t1.at[iv1.at[j]], r1[b], s1[b]),
                       pltpu.async_copy(t2.at[iv2.at[j]], r2[b], s2[b])))
        for b in range(_K):
            j = k * _K + b
            h1, h2 = hs[b]
            h1.wait()
            pltpu.sync_copy(r1[b], o1.at[pl.ds(base + j * _CH, _CH), :])
            h2.wait()
            pltpu.sync_copy(r2[b], o2.at[pl.ds(base + j * _CH, _CH), :])
        return carry

    lax.fori_loop(0, _NCHUNK // _K, body, 0)
    for t in range(_NCHUNK % _K):
        jt = (_NCHUNK // _K) * _K + t
        h1 = pltpu.async_copy(t1.at[iv1.at[jt]], r1[0], s1[0])
        h2 = pltpu.async_copy(t2.at[iv2.at[jt]], r2[0], s2[0])
        h1.wait()
        pltpu.sync_copy(r1[0], o1.at[pl.ds(base + jt * _CH, _CH), :])
        h2.wait()
        pltpu.sync_copy(r2[0], o2.at[pl.ds(base + jt * _CH, _CH), :])


@functools.partial(
    pl.kernel,
    mesh=_sc_mesh(),
    out_type=jax.ShapeDtypeStruct((_NC, N, _PC), jnp.float32),
    scratch_types=(
        [pltpu.VMEM((1, _CH), jnp.int32)] * _KS
        + [pltpu.VMEM((_CH, _PC), jnp.float32)] * _KS
        + [pltpu.VMEM_SHARED((_NACC, _PC), jnp.float32)]
        + [pltpu.SemaphoreType.DMA] * (2 * _KS)
    ),
)
def _sc_scatter_add(pay, idx, zer, out, *bufs):
    ivr = bufs[0:_KS]
    pv = bufs[_KS:2 * _KS]
    acc = bufs[2 * _KS]
    si = bufs[2 * _KS + 1:3 * _KS + 1]
    sp = bufs[3 * _KS + 1:4 * _KS + 1]
    cid = lax.axis_index("c")
    sid = lax.axis_index("s")

    pltpu.sync_copy(zer, acc.at[pl.ds(sid * _RPT, _RPT), :])
    plsc.subcore_barrier()

    def scatter_all():
        def wbody(s, carry):
            w = s * _NC + cid
            base = w * _PER

            def chunk_group(k, nb):
                hs = []
                for b in range(nb):
                    j = k * _KS + b
                    hs.append(
                        (pltpu.async_copy(idx.at[w, j], ivr[b], si[b]),
                         pltpu.async_copy(
                             pay.at[pl.ds(base + j * _CH, _CH), :],
                             pv[b], sp[b])))
                for b in range(nb):
                    hi, hp = hs[b]
                    hi.wait()
                    hp.wait()
                    pltpu.sync_copy(pv[b], acc.at[ivr[b].at[0]], add=True)

            def body(k, carry2):
                chunk_group(k, _KS)
                return carry2

            lax.fori_loop(0, _NCHUNK // _KS, body, 0)
            for t in range(_NCHUNK % _KS):
                jt = (_NCHUNK // _KS) * _KS + t
                hi = pltpu.async_copy(idx.at[w, jt], ivr[0], si[0])
                hp = pltpu.async_copy(
                    pay.at[pl.ds(base + jt * _CH, _CH), :], pv[0], sp[0])
                hi.wait()
                hp.wait()
                pltpu.sync_copy(pv[0], acc.at[ivr[0].at[0]], add=True)
            return carry

        lax.fori_loop(0, _NS, wbody, 0)

    @pl.when(sid == 0)
    def _scatter_half():
        scatter_all()

    plsc.subcore_barrier()

    @pl.when(sid < _NS - 1)
    def _dump_full():
        pltpu.sync_copy(acc.at[pl.ds(sid * _RPT, _RPT), :],
                        out.at[cid, pl.ds(sid * _RPT, _RPT), :])

    @pl.when(sid == _NS - 1)
    def _dump_tail():
        pltpu.sync_copy(acc.at[pl.ds(sid * _RPT, _DPT), :],
                        out.at[cid, pl.ds(sid * _RPT, _DPT), :])


def _embed_kernel(f_ref, x_ref, w8, br, b316, ver, qw1, qb1, qw2, qb2,
                  s_out, v_out, q_out):
    f = f_ref[...]
    s0 = f @ w8[...] + br[...]
    v0 = (x_ref[...] @ b316[...]) * ver[...]
    q0 = _mlp2k(s0, qw1[...], qb1[...], qw2[...], qb2[...])
    s_out[...] = s0
    v_out[...] = v0
    q_out[...] = q0


def _aux_kernel(xs_ref, xd_ref, ea_ref, aux_out):
    xs = xs_ref[...]
    xd = xd_ref[...]
    rel = xs[:, 0:3] - xd[:, 0:3]
    d2 = jnp.sum(rel * rel, axis=-1, keepdims=True) + 1e-12
    dist = jnp.sqrt(d2)
    ru = rel / (dist + 1e-8)
    ea = ea_ref[...][:, 0:5]
    zeros = jnp.zeros((ea.shape[0], 7), jnp.float32)
    aux_out[...] = jnp.concatenate([ea, dist, ru, zeros], axis=-1)


def _edge_kernel(s_ref, q_ref, aux_ref,
                 wk1a, wk1b, kb1, kw2, kb2,
                 wv1a, wv1b, vb1, vw2, vb2,
                 gw, gb, sel, selt, a24, b16, out_ref, out2_ref):
    s = s_ref[...]
    qd = q_ref[...]
    aux = aux_ref[...]
    hk = jax.nn.relu(_lnk(s @ wk1a[...] + aux @ wk1b[...] + kb1[...]))
    k = hk @ kw2[...] + kb2[...]
    hv = jax.nn.relu(_lnk(s @ wv1a[...] + aux @ wv1b[...] + vb1[...]))
    v = hv @ vw2[...] + vb2[...]
    logits = ((qd * k) @ sel[...]) * _INV_SQRT_DH
    ex = jnp.exp(logits)
    wv = v * (ex @ selt[...])
    gate = jnp.tanh(v @ gw[...] + gb[...])
    vm = (gate @ a24[...]) * (aux @ b16[...])
    zeros = jnp.zeros((s.shape[0], 100), jnp.float32)
    out_ref[...] = wv
    out2_ref[...] = jnp.concatenate([ex, vm, zeros], axis=-1)


def _node_kernel(part_ref, part2_ref, s_ref, v_ref,
                 selt, ow1, ob1, ow2, ob2, vsr, a24, a24t,
                 qw1, qb1, qw2, qb2,
                 s_out, v_out, q_out):
    pr = part_ref[...]
    numer = pr[0] + pr[1]
    pr2 = part2_ref[...]
    sall2 = pr2[0] + pr2[1]
    den4 = sall2[:, 0:4]
    vagg = sall2[:, 4:28]
    agg = numer / (den4 @ selt[...] + 1e-9)
    upd = _mlp2k(agg, ow1[...], ob1[...], ow2[...], ob2[...])
    s1 = _lnk(s_ref[...] + upd)
    vr = v_ref[...] + vagg
    v2 = (vr * vr) @ a24t[...]
    vnorm = jnp.sqrt(v2 + 1e-12)
    fac = vsr[...] / (1.0 + vnorm)
    v1 = vr * (fac @ a24[...])
    qn = _mlp2k(s1, qw1[...], qb1[...], qw2[...], qb2[...])
    s_out[...] = s1
    v_out[...] = v1
    q_out[...] = qn


def _feat_kernel(s_ref, v_ref, a24t, w1a, w1b, b1, w2, b2, f_out):
    sc = s_ref[...]
    vr = v_ref[...]
    vinv = jnp.sqrt((vr * vr) @ a24t[...] + 1e-12)
    h = jax.nn.relu(_lnk(sc @ w1a[...] + vinv @ w1b[...] + b1[...]))
    f_out[...] = h @ w2[...] + b2[...] + sc


def _pool_kernel(ids_ref, f_ref, gw1, gb1, gw2, gb2, out_ref):
    step = pl.program_id(0)

    @pl.when(step == 0)
    def _init():
        out_ref[...] = jnp.full((G, D), -1e30, jnp.float32)

    def body(r, carry):
        g = ids_ref[0, 0, r]
        row = f_ref[pl.ds(r, 1), :]
        out_ref[pl.ds(g, 1), :] = jnp.maximum(out_ref[pl.ds(g, 1), :], row)
        return carry

    lax.fori_loop(0, _BN, body, 0)

    @pl.when(step == pl.num_programs(0) - 1)
    def _head():
        pooled = out_ref[...]
        pooled = jnp.where(pooled == -1e30, 0.0, pooled)
        h = jax.nn.relu(_lnk(pooled @ gw1[...] + gb1[...]))
        out_ref[...] = h @ gw2[...] + gb2[...]


def _full(shape):
    nd = len(shape)
    return pl.BlockSpec(shape, lambda i, _nd=nd: (0,) * _nd)


def _rows(block, ncols):
    return pl.BlockSpec((block, ncols), lambda i: (i, 0))


def _tc_embed(f8, x16, w8, br, b316, ver, qw1, qb1, qw2, qb2):
    return pl.pallas_call(
        _embed_kernel,
        grid=(N // _BN,),
        in_specs=[_rows(_BN, 8), _rows(_BN, 16),
                  _full((8, D)), _full((1, D)), _full((16, 24)), _full((1, 24)),
                  _full((D, D)), _full((1, D)), _full((D, D)), _full((1, D))],
        out_specs=[_rows(_BN, D), _rows(_BN, 24), _rows(_BN, D)],
        out_shape=[jax.ShapeDtypeStruct((N, D), jnp.float32),
                   jax.ShapeDtypeStruct((N, 24), jnp.float32),
                   jax.ShapeDtypeStruct((N, D), jnp.float32)],
    )(f8, x16, w8, br, b316, ver, qw1, qb1, qw2, qb2)


def _tc_aux(xs16, xd16, ea8):
    return pl.pallas_call(
        _aux_kernel,
        grid=(E // _BE,),
        in_specs=[_rows(_BE, D), _rows(_BE, D), _rows(_BE, 8)],
        out_specs=_rows(_BE, 16),
        out_shape=jax.ShapeDtypeStruct((E, 16), jnp.float32),
    )(xs16, xd16, ea8)


def _tc_edge(ssrc, qdst, aux, wk1a, wk1b, kb1, kw2, kb2,
             wv1a, wv1b, vb1, vw2, vb2, gw, gb, sel, selt, a24, b16):
    return pl.pallas_call(
        _edge_kernel,
        grid=(E // _BE,),
        in_specs=[_rows(_BE, D), _rows(_BE, D), _rows(_BE, 16),
                  _full((D, D)), _full((16, D)), _full((1, D)),
                  _full((D, D)), _full((1, D)),
                  _full((D, D)), _full((16, D)), _full((1, D)),
                  _full((D, D)), _full((1, D)),
                  _full((D, VC)), _full((1, VC)),
                  _full((D, H)), _full((H, D)),
                  _full((VC, 24)), _full((16, 24))],
        out_specs=[_rows(_BE, _PC), _rows(_BE, _PC)],
        out_shape=[jax.ShapeDtypeStruct((E, _PC), jnp.float32),
                   jax.ShapeDtypeStruct((E, _PC), jnp.float32)],
    )(ssrc, qdst, aux, wk1a, wk1b, kb1, kw2, kb2,
      wv1a, wv1b, vb1, vw2, vb2, gw, gb, sel, selt, a24, b16)


def _tc_node(part, part2, scalar, vec, selt, ow1, ob1, ow2, ob2, vsr, a24, a24t,
             qw1, qb1, qw2, qb2):
    return pl.pallas_call(
        _node_kernel,
        grid=(N // _BN,),
        in_specs=[pl.BlockSpec((_NC, _BN, _PC), lambda i: (0, i, 0)),
                  pl.BlockSpec((_NC, _BN, _PC), lambda i: (0, i, 0)),
                  _rows(_BN, D), _rows(_BN, 24),
                  _full((H, D)),
                  _full((D, D)), _full((1, D)), _full((D, D)), _full((1, D)),
                  _full((1, VC)), _full((VC, 24)), _full((24, VC)),
                  _full((D, D)), _full((1, D)), _full((D, D)), _full((1, D))],
        out_specs=[_rows(_BN, D), _rows(_BN, 24), _rows(_BN, D)],
        out_shape=[jax.ShapeDtypeStruct((N, D), jnp.float32),
                   jax.ShapeDtypeStruct((N, 24), jnp.float32),
                   jax.ShapeDtypeStruct((N, D), jnp.float32)],
    )(part, part2, scalar, vec, selt, ow1, ob1, ow2, ob2, vsr, a24, a24t,
      qw1, qb1, qw2, qb2)


def _tc_feat(scalar, vec, a24t, w1a, w1b, b1, w2, b2):
    return pl.pallas_call(
        _feat_kernel,
        grid=(N // _BN,),
        in_specs=[_rows(_BN, D), _rows(_BN, 24), _full((24, VC)),
                  _full((D, D)), _full((VC, D)), _full((1, D)),
                  _full((D, D)), _full((1, D))],
        out_specs=_rows(_BN, D),
        out_shape=jax.ShapeDtypeStruct((N, D), jnp.float32),
    )(scalar, vec, a24t, w1a, w1b, b1, w2, b2)


def _tc_pool(ids3, feat, gw1, gb1, gw2, gb2):
    return pl.pallas_call(
        _pool_kernel,
        grid=(N // _BN,),
        in_specs=[pl.BlockSpec((1, 1, _BN), lambda i: (i, 0, 0),
                               memory_space=pltpu.SMEM),
                  _rows(_BN, D),
                  _full((D, D)), _full((1, D)), _full((D, D)), _full((1, D))],
        out_specs=_full((G, D)),
        out_shape=jax.ShapeDtypeStruct((G, D), jnp.float32),
    )(ids3, feat, gw1, gb1, gw2, gb2)


def kernel(node_f, node_x, edge_index, edge_attr, graph_ids, params):
    p = params
    src2 = edge_index[0].reshape(_NW, _IPW, _CH).astype(jnp.int32)
    dst2 = edge_index[1].reshape(_NW, _IPW, _CH).astype(jnp.int32)
    dst4 = dst2.reshape(_NW, _IPW, 1, _CH)

    f8 = jnp.pad(node_f[:, :, 0], ((0, 0), (0, 2)))
    x16 = jnp.pad(node_x, ((0, 0), (0, 13)))
    ea8 = jnp.pad(edge_attr, ((0, 0), (0, 3)))
    zer = jnp.zeros((_RPT, _PC), jnp.float32)

    sel = jnp.asarray(_SEL)
    selt = jnp.asarray(_SELT)
    a24 = jnp.asarray(_A24)
    a24t = jnp.asarray(_A24T)
    b16 = jnp.asarray(_B16)
    b316 = jnp.asarray(_B3_16)

    embw = p['embed_W']
    embw = embw.at[5].set(embw[5] / 9.0)
    w8 = jnp.concatenate([embw, jnp.zeros((2, D), jnp.float32)], axis=0)
    br = p['embed_b'][None, :]
    ver = jnp.repeat(p['vec_embed'][0], 3)[None, :]

    lps = [p['l%d' % l] for l in range(4)]
    lp0 = lps[0]
    scalar, vec, q = _tc_embed(
        f8, x16, w8, br, b316, ver,
        lp0['qW1'], lp0['qb1'][None, :], lp0['qW2'], lp0['qb2'][None, :])

    x128 = jnp.pad(node_x, ((0, 0), (0, D - 3)))
    xs, xd = _sc_gather_pair(x128, src2, x128, dst2)
    aux = _tc_aux(xs, xd, ea8)

    for l in range(4):
        lp = lps[l]
        wk1b = jnp.concatenate(
            [lp['kW1'][D:D + 6], jnp.zeros((10, D), jnp.float32)], axis=0)
        wv1b = jnp.concatenate(
            [lp['vW1'][D:D + 6], jnp.zeros((10, D), jnp.float32)], axis=0)

        ssrc, qdst = _sc_gather_pair(scalar, src2, q, dst2)
        pay, pay2 = _tc_edge(
            ssrc, qdst, aux,
            lp['kW1'][:D], wk1b, lp['kb1'][None, :], lp['kW2'], lp['kb2'][None, :],
            lp['vW1'][:D], wv1b, lp['vb1'][None, :], lp['vW2'], lp['vb2'][None, :],
            lp['gW'], lp['gb'][None, :], sel, selt, a24, b16)
        part = _sc_scatter_add(pay, dst4, zer)
        part2 = _sc_scatter_add(pay2, dst4, zer)

        lpn = lps[(l + 1) % 4]
        scalar, vec, q = _tc_node(
            part, part2, scalar, vec, selt,
            lp['oW1'], lp['ob1'][None, :], lp['oW2'], lp['ob2'][None, :],
            lp['vscale'][None, :], a24, a24t,
            lpn['qW1'], lpn['qb1'][None, :], lpn['qW2'], lpn['qb2'][None, :])

    nmw1b = p['nmW1'][D:D + VC]
    feat = _tc_feat(scalar, vec, a24t,
                    p['nmW1'][:D], nmw1b, p['nmb1'][None, :],
                    p['nmW2'], p['nmb2'][None, :])

    gw2 = jnp.zeros((D, D), jnp.float32).at[:, 0:1].set(p['gmW2'])
    gb2 = jnp.zeros((1, D), jnp.float32).at[0, 0].set(p['gmb2'][0])
    ids3 = graph_ids.reshape(N // _BN, 1, _BN).astype(jnp.int32)
    out = _tc_pool(ids3, feat, p['gmW1'], p['gmb1'][None, :], gw2, gb2)
    return out[:, 0:1]


# speedup vs baseline: 7.9466x; 1.0307x over previous
import functools

import jax
import jax.numpy as jnp
import numpy as np
from jax import lax
from jax.experimental import pallas as pl
from jax.experimental.pallas import tpu as pltpu
from jax.experimental.pallas import tpu_sc as plsc

N = 10000
E = 320000
D = 128
H = 4
DH = 32
VC = 8
G = 512

_NC, _NS = 2, 16
_NW = _NC * _NS
_CH = 80
_PER = E // _NW
_NCHUNK = _PER // _CH
_IDXROWS = E // _CH
_IPW = _IDXROWS // _NW
_NACC = 10240
_RPT = _NACC // _NS
_DPT = 400
_BN = 1000
_K = 4
_KS = 2
_BE = 2000
_PC = 128

_INV_SQRT_DH = float(1.0 / np.sqrt(DH))

_SEL = np.repeat(np.eye(H, dtype=np.float32), DH, axis=0)
_SELT = np.ascontiguousarray(_SEL.T)
_A24 = np.kron(np.eye(VC, dtype=np.float32), np.ones((1, 3), np.float32))
_A24T = np.ascontiguousarray(_A24.T)
_B3 = np.kron(np.ones((1, VC), np.float32), np.eye(3, dtype=np.float32))
_B16 = np.zeros((16, 24), np.float32)
_B16[6:9] = _B3
_B3_16 = np.zeros((16, 24), np.float32)
_B3_16[0:3] = _B3


def _lnk(x):
    m = jnp.mean(x, axis=-1, keepdims=True)
    xc = x - m
    v = jnp.mean(xc * xc, axis=-1, keepdims=True)
    return xc / jnp.sqrt(v + 1e-5)


def _mlp2k(x, w1, b1, w2, b2):
    h = jax.nn.relu(_lnk(x @ w1 + b1))
    return h @ w2 + b2


def _sc_mesh():
    return plsc.VectorSubcoreMesh(core_axis_name="c", subcore_axis_name="s")


@functools.partial(
    pl.kernel,
    mesh=_sc_mesh(),
    out_type=(jax.ShapeDtypeStruct((E, D), jnp.float32),
              jax.ShapeDtypeStruct((E, D), jnp.float32)),
    scratch_types=(
        [pltpu.VMEM((_IPW, _CH), jnp.int32),
         pltpu.VMEM((_IPW, _CH), jnp.int32)]
        + [pltpu.VMEM((_CH, D), jnp.float32)] * (2 * _K)
        + [pltpu.SemaphoreType.DMA] * (4 * _K)
    ),
)
def _sc_gather_pair(t1, i1, t2, i2, o1, o2, iv1, iv2, *bufs):
    r1 = bufs[0:_K]
    r2 = bufs[_K:2 * _K]
    s1 = bufs[2 * _K:3 * _K]
    s2 = bufs[3 * _K:4 * _K]
    w1 = bufs[4 * _K:5 * _K]
    w2 = bufs[5 * _K:6 * _K]
    wid = lax.axis_index("s") * _NC + lax.axis_index("c")
    base = wid * _PER
    pltpu.sync_copy(i1.at[wid], iv1)
    pltpu.sync_copy(i2.at[wid], iv2)

    def body(k, carry):
        hs = []
        for b in range(_K):
            j = k * _K + b
            hs.append((pltpu.async_copy(t1.at[iv1.at[j]], r1[b], s1[b]),
                       pltpu.async_copy(t2.at[iv2.at[j]], r2[b], s2[b])))
        ws = []
        for b in range(_K):
            j = k * _K + b
            h1, h2 = hs[b]
            h1.wait()
            ws.append(pltpu.async_copy(
                r1[b], o1.at[pl.ds(base + j * _CH, _CH), :], w1[b]))
            h2.wait()
            ws.append(pltpu.async_copy(
                r2[b], o2.at[pl.ds(base + j * _CH, _CH), :], w2[b]))
        for h in ws:
            h.wait()
        return carry

    lax.fori_loop(0, _NCHUNK // _K, body, 0)
    for t in range(_NCHUNK % _K):
        jt = (_NCHUNK // _K) * _K + t
        h1 = pltpu.async_copy(t1.at[iv1.at[jt]], r1[0], s1[0])
        h2 = pltpu.async_copy(t2.at[iv2.at[jt]], r2[0], s2[0])
        h1.wait()
        pltpu.sync_copy(r1[0], o1.at[pl.ds(base + jt * _CH, _CH), :])
        h2.wait()
        pltpu.sync_copy(r2[0], o2.at[pl.ds(base + jt * _CH, _CH), :])


@functools.partial(
    pl.kernel,
    mesh=_sc_mesh(),
    out_type=jax.ShapeDtypeStruct((_NC, N, _PC), jnp.float32),
    scratch_types=(
        [pltpu.VMEM((1, _CH), jnp.int32)] * _KS
        + [pltpu.VMEM((_CH, _PC), jnp.float32)] * _KS
        + [pltpu.VMEM_SHARED((_NACC, _PC), jnp.float32)]
        + [pltpu.SemaphoreType.DMA] * (3 * _KS)
    ),
)
def _sc_scatter_add(pay, idx, zer, out, *bufs):
    ivr = bufs[0:_KS]
    pv = bufs[_KS:2 * _KS]
    acc = bufs[2 * _KS]
    si = bufs[2 * _KS + 1:3 * _KS + 1]
    sp = bufs[3 * _KS + 1:4 * _KS + 1]
    sa = bufs[4 * _KS + 1:5 * _KS + 1]
    cid = lax.axis_index("c")
    sid = lax.axis_index("s")

    pltpu.sync_copy(zer, acc.at[pl.ds(sid * _RPT, _RPT), :])
    plsc.subcore_barrier()

    def scatter_all():
        def wbody(s, carry):
            w = s * _NC + cid
            base = w * _PER

            def chunk_group(k, nb):
                hs = []
                for b in range(nb):
                    j = k * _KS + b
                    hs.append(
                        (pltpu.async_copy(idx.at[w, j], ivr[b], si[b]),
                         pltpu.async_copy(
                             pay.at[pl.ds(base + j * _CH, _CH), :],
                             pv[b], sp[b])))
                has = []
                for b in range(nb):
                    hi, hp = hs[b]
                    hi.wait()
                    hp.wait()
                    has.append(pltpu.async_copy(
                        pv[b], acc.at[ivr[b].at[0]], sa[b], add=True))
                for h in has:
                    h.wait()

            def body(k, carry2):
                chunk_group(k, _KS)
                return carry2

            lax.fori_loop(0, _NCHUNK // _KS, body, 0)
            for t in range(_NCHUNK % _KS):
                jt = (_NCHUNK // _KS) * _KS + t
                hi = pltpu.async_copy(idx.at[w, jt], ivr[0], si[0])
                hp = pltpu.async_copy(
                    pay.at[pl.ds(base + jt * _CH, _CH), :], pv[0], sp[0])
                hi.wait()
                hp.wait()
                pltpu.sync_copy(pv[0], acc.at[ivr[0].at[0]], add=True)
            return carry

        lax.fori_loop(0, _NS, wbody, 0)

    @pl.when(sid == 0)
    def _scatter_half():
        scatter_all()

    plsc.subcore_barrier()

    @pl.when(sid < _NS - 1)
    def _dump_full():
        pltpu.sync_copy(acc.at[pl.ds(sid * _RPT, _RPT), :],
                        out.at[cid, pl.ds(sid * _RPT, _RPT), :])

    @pl.when(sid == _NS - 1)
    def _dump_tail():
        pltpu.sync_copy(acc.at[pl.ds(sid * _RPT, _DPT), :],
                        out.at[cid, pl.ds(sid * _RPT, _DPT), :])


def _embed_kernel(f_ref, x_ref, w8, br, b316, ver, qw1, qb1, qw2, qb2,
                  s_out, v_out, q_out):
    f = f_ref[...]
    s0 = f @ w8[...] + br[...]
    v0 = (x_ref[...] @ b316[...]) * ver[...]
    q0 = _mlp2k(s0, qw1[...], qb1[...], qw2[...], qb2[...])
    s_out[...] = s0
    v_out[...] = v0
    q_out[...] = q0


def _aux_kernel(xs_ref, xd_ref, ea_ref, aux_out):
    xs = xs_ref[...]
    xd = xd_ref[...]
    rel = xs[:, 0:3] - xd[:, 0:3]
    d2 = jnp.sum(rel * rel, axis=-1, keepdims=True) + 1e-12
    dist = jnp.sqrt(d2)
    ru = rel / (dist + 1e-8)
    ea = ea_ref[...][:, 0:5]
    zeros = jnp.zeros((ea.shape[0], 7), jnp.float32)
    aux_out[...] = jnp.concatenate([ea, dist, ru, zeros], axis=-1)


def _edge_kernel(s_ref, q_ref, aux_ref,
                 wk1a, wk1b, kb1, kw2, kb2,
                 wv1a, wv1b, vb1, vw2, vb2,
                 gw, gb, sel, selt, a24, b16, out_ref, out2_ref):
    s = s_ref[...]
    qd = q_ref[...]
    aux = aux_ref[...]
    hk = jax.nn.relu(_lnk(s @ wk1a[...] + aux @ wk1b[...] + kb1[...]))
    k = hk @ kw2[...] + kb2[...]
    hv = jax.nn.relu(_lnk(s @ wv1a[...] + aux @ wv1b[...] + vb1[...]))
    v = hv @ vw2[...] + vb2[...]
    logits = ((qd * k) @ sel[...]) * _INV_SQRT_DH
    ex = jnp.exp(logits)
    wv = v * (ex @ selt[...])
    gate = jnp.tanh(v @ gw[...] + gb[...])
    vm = (gate @ a24[...]) * (aux @ b16[...])
    zeros = jnp.zeros((s.shape[0], 100), jnp.float32)
    out_ref[...] = wv
    out2_ref[...] = jnp.concatenate([ex, vm, zeros], axis=-1)


def _node_kernel(part_ref, part2_ref, s_ref, v_ref,
                 selt, ow1, ob1, ow2, ob2, vsr, a24, a24t,
                 qw1, qb1, qw2, qb2,
                 s_out, v_out, q_out):
    pr = part_ref[...]
    numer = pr[0] + pr[1]
    pr2 = part2_ref[...]
    sall2 = pr2[0] + pr2[1]
    den4 = sall2[:, 0:4]
    vagg = sall2[:, 4:28]
    agg = numer / (den4 @ selt[...] + 1e-9)
    upd = _mlp2k(agg, ow1[...], ob1[...], ow2[...], ob2[...])
    s1 = _lnk(s_ref[...] + upd)
    vr = v_ref[...] + vagg
    v2 = (vr * vr) @ a24t[...]
    vnorm = jnp.sqrt(v2 + 1e-12)
    fac = vsr[...] / (1.0 + vnorm)
    v1 = vr * (fac @ a24[...])
    qn = _mlp2k(s1, qw1[...], qb1[...], qw2[...], qb2[...])
    s_out[...] = s1
    v_out[...] = v1
    q_out[...] = qn


def _feat_kernel(s_ref, v_ref, a24t, w1a, w1b, b1, w2, b2, f_out):
    sc = s_ref[...]
    vr = v_ref[...]
    vinv = jnp.sqrt((vr * vr) @ a24t[...] + 1e-12)
    h = jax.nn.relu(_lnk(sc @ w1a[...] + vinv @ w1b[...] + b1[...]))
    f_out[...] = h @ w2[...] + b2[...] + sc


def _pool_kernel(ids_ref, f_ref, gw1, gb1, gw2, gb2, out_ref):
    step = pl.program_id(0)

    @pl.when(step == 0)
    def _init():
        out_ref[...] = jnp.full((G, D), -1e30, jnp.float32)

    def body(r, carry):
        g = ids_ref[0, 0, r]
        row = f_ref[pl.ds(r, 1), :]
        out_ref[pl.ds(g, 1), :] = jnp.maximum(out_ref[pl.ds(g, 1), :], row)
        return carry

    lax.fori_loop(0, _BN, body, 0)

    @pl.when(step == pl.num_programs(0) - 1)
    def _head():
        pooled = out_ref[...]
        pooled = jnp.where(pooled == -1e30, 0.0, pooled)
        h = jax.nn.relu(_lnk(pooled @ gw1[...] + gb1[...]))
        out_ref[...] = h @ gw2[...] + gb2[...]


def _full(shape):
    nd = len(shape)
    return pl.BlockSpec(shape, lambda i, _nd=nd: (0,) * _nd)


def _rows(block, ncols):
    return pl.BlockSpec((block, ncols), lambda i: (i, 0))


def _tc_embed(f8, x16, w8, br, b316, ver, qw1, qb1, qw2, qb2):
    return pl.pallas_call(
        _embed_kernel,
        grid=(N // _BN,),
        in_specs=[_rows(_BN, 8), _rows(_BN, 16),
                  _full((8, D)), _full((1, D)), _full((16, 24)), _full((1, 24)),
                  _full((D, D)), _full((1, D)), _full((D, D)), _full((1, D))],
        out_specs=[_rows(_BN, D), _rows(_BN, 24), _rows(_BN, D)],
        out_shape=[jax.ShapeDtypeStruct((N, D), jnp.float32),
                   jax.ShapeDtypeStruct((N, 24), jnp.float32),
                   jax.ShapeDtypeStruct((N, D), jnp.float32)],
    )(f8, x16, w8, br, b316, ver, qw1, qb1, qw2, qb2)


def _tc_aux(xs16, xd16, ea8):
    return pl.pallas_call(
        _aux_kernel,
        grid=(E // _BE,),
        in_specs=[_rows(_BE, D), _rows(_BE, D), _rows(_BE, 8)],
        out_specs=_rows(_BE, 16),
        out_shape=jax.ShapeDtypeStruct((E, 16), jnp.float32),
    )(xs16, xd16, ea8)


def _tc_edge(ssrc, qdst, aux, wk1a, wk1b, kb1, kw2, kb2,
             wv1a, wv1b, vb1, vw2, vb2, gw, gb, sel, selt, a24, b16):
    return pl.pallas_call(
        _edge_kernel,
        grid=(E // _BE,),
        in_specs=[_rows(_BE, D), _rows(_BE, D), _rows(_BE, 16),
                  _full((D, D)), _full((16, D)), _full((1, D)),
                  _full((D, D)), _full((1, D)),
                  _full((D, D)), _full((16, D)), _full((1, D)),
                  _full((D, D)), _full((1, D)),
                  _full((D, VC)), _full((1, VC)),
                  _full((D, H)), _full((H, D)),
                  _full((VC, 24)), _full((16, 24))],
        out_specs=[_rows(_BE, _PC), _rows(_BE, _PC)],
        out_shape=[jax.ShapeDtypeStruct((E, _PC), jnp.float32),
                   jax.ShapeDtypeStruct((E, _PC), jnp.float32)],
    )(ssrc, qdst, aux, wk1a, wk1b, kb1, kw2, kb2,
      wv1a, wv1b, vb1, vw2, vb2, gw, gb, sel, selt, a24, b16)


def _tc_node(part, part2, scalar, vec, selt, ow1, ob1, ow2, ob2, vsr, a24, a24t,
             qw1, qb1, qw2, qb2):
    return pl.pallas_call(
        _node_kernel,
        grid=(N // _BN,),
        in_specs=[pl.BlockSpec((_NC, _BN, _PC), lambda i: (0, i, 0)),
                  pl.BlockSpec((_NC, _BN, _PC), lambda i: (0, i, 0)),
                  _rows(_BN, D), _rows(_BN, 24),
                  _full((H, D)),
                  _full((D, D)), _full((1, D)), _full((D, D)), _full((1, D)),
                  _full((1, VC)), _full((VC, 24)), _full((24, VC)),
                  _full((D, D)), _full((1, D)), _full((D, D)), _full((1, D))],
        out_specs=[_rows(_BN, D), _rows(_BN, 24), _rows(_BN, D)],
        out_shape=[jax.ShapeDtypeStruct((N, D), jnp.float32),
                   jax.ShapeDtypeStruct((N, 24), jnp.float32),
                   jax.ShapeDtypeStruct((N, D), jnp.float32)],
    )(part, part2, scalar, vec, selt, ow1, ob1, ow2, ob2, vsr, a24, a24t,
      qw1, qb1, qw2, qb2)


def _tc_feat(scalar, vec, a24t, w1a, w1b, b1, w2, b2):
    return pl.pallas_call(
        _feat_kernel,
        grid=(N // _BN,),
        in_specs=[_rows(_BN, D), _rows(_BN, 24), _full((24, VC)),
                  _full((D, D)), _full((VC, D)), _full((1, D)),
                  _full((D, D)), _full((1, D))],
        out_specs=_rows(_BN, D),
        out_shape=jax.ShapeDtypeStruct((N, D), jnp.float32),
    )(scalar, vec, a24t, w1a, w1b, b1, w2, b2)


def _tc_pool(ids3, feat, gw1, gb1, gw2, gb2):
    return pl.pallas_call(
        _pool_kernel,
        grid=(N // _BN,),
        in_specs=[pl.BlockSpec((1, 1, _BN), lambda i: (i, 0, 0),
                               memory_space=pltpu.SMEM),
                  _rows(_BN, D),
                  _full((D, D)), _full((1, D)), _full((D, D)), _full((1, D))],
        out_specs=_full((G, D)),
        out_shape=jax.ShapeDtypeStruct((G, D), jnp.float32),
    )(ids3, feat, gw1, gb1, gw2, gb2)


def kernel(node_f, node_x, edge_index, edge_attr, graph_ids, params):
    p = params
    src2 = edge_index[0].reshape(_NW, _IPW, _CH).astype(jnp.int32)
    dst2 = edge_index[1].reshape(_NW, _IPW, _CH).astype(jnp.int32)
    dst4 = dst2.reshape(_NW, _IPW, 1, _CH)

    f8 = jnp.pad(node_f[:, :, 0], ((0, 0), (0, 2)))
    x16 = jnp.pad(node_x, ((0, 0), (0, 13)))
    ea8 = jnp.pad(edge_attr, ((0, 0), (0, 3)))
    zer = jnp.zeros((_RPT, _PC), jnp.float32)

    sel = jnp.asarray(_SEL)
    selt = jnp.asarray(_SELT)
    a24 = jnp.asarray(_A24)
    a24t = jnp.asarray(_A24T)
    b16 = jnp.asarray(_B16)
    b316 = jnp.asarray(_B3_16)

    embw = p['embed_W']
    embw = embw.at[5].set(embw[5] / 9.0)
    w8 = jnp.concatenate([embw, jnp.zeros((2, D), jnp.float32)], axis=0)
    br = p['embed_b'][None, :]
    ver = jnp.repeat(p['vec_embed'][0], 3)[None, :]

    lps = [p['l%d' % l] for l in range(4)]
    lp0 = lps[0]
    scalar, vec, q = _tc_embed(
        f8, x16, w8, br, b316, ver,
        lp0['qW1'], lp0['qb1'][None, :], lp0['qW2'], lp0['qb2'][None, :])

    x128 = jnp.pad(node_x, ((0, 0), (0, D - 3)))
    xs, xd = _sc_gather_pair(x128, src2, x128, dst2)
    aux = _tc_aux(xs, xd, ea8)

    for l in range(4):
        lp = lps[l]
        wk1b = jnp.concatenate(
            [lp['kW1'][D:D + 6], jnp.zeros((10, D), jnp.float32)], axis=0)
        wv1b = jnp.concatenate(
            [lp['vW1'][D:D + 6], jnp.zeros((10, D), jnp.float32)], axis=0)

        ssrc, qdst = _sc_gather_pair(scalar, src2, q, dst2)
        pay, pay2 = _tc_edge(
            ssrc, qdst, aux,
            lp['kW1'][:D], wk1b, lp['kb1'][None, :], lp['kW2'], lp['kb2'][None, :],
            lp['vW1'][:D], wv1b, lp['vb1'][None, :], lp['vW2'], lp['vb2'][None, :],
            lp['gW'], lp['gb'][None, :], sel, selt, a24, b16)
        part = _sc_scatter_add(pay, dst4, zer)
        part2 = _sc_scatter_add(pay2, dst4, zer)

        lpn = lps[(l + 1) % 4]
        scalar, vec, q = _tc_node(
            part, part2, scalar, vec, selt,
            lp['oW1'], lp['ob1'][None, :], lp['oW2'], lp['ob2'][None, :],
            lp['vscale'][None, :], a24, a24t,
            lpn['qW1'], lpn['qb1'][None, :], lpn['qW2'], lpn['qb2'][None, :])

    nmw1b = p['nmW1'][D:D + VC]
    feat = _tc_feat(scalar, vec, a24t,
                    p['nmW1'][:D], nmw1b, p['nmb1'][None, :],
                    p['nmW2'], p['nmb2'][None, :])

    gw2 = jnp.zeros((D, D), jnp.float32).at[:, 0:1].set(p['gmW2'])
    gb2 = jnp.zeros((1, D), jnp.float32).at[0, 0].set(p['gmb2'][0])
    ids3 = graph_ids.reshape(N // _BN, 1, _BN).astype(jnp.int32)
    out = _tc_pool(ids3, feat, p['gmW1'], p['gmb1'][None, :], gw2, gb2)
    return out[:, 0:1]


# speedup vs baseline: 7.9731x; 1.0033x over previous
import functools

import jax
import jax.numpy as jnp
import numpy as np
from jax import lax
from jax.experimental import pallas as pl
from jax.experimental.pallas import tpu as pltpu
from jax.experimental.pallas import tpu_sc as plsc

N = 10000
E = 320000
D = 128
H = 4
DH = 32
VC = 8
G = 512

_NC, _NS = 2, 16
_NW = _NC * _NS
_CH = 80
_PER = E // _NW
_NCHUNK = _PER // _CH
_IDXROWS = E // _CH
_IPW = _IDXROWS // _NW
_NACC = 10240
_RPT = _NACC // _NS
_DPT = 400
_BN = 1000
_K = 4
_KS = 2
_BE = 2000
_PC = 128

_INV_SQRT_DH = float(1.0 / np.sqrt(DH))

_SEL = np.repeat(np.eye(H, dtype=np.float32), DH, axis=0)
_SELT = np.ascontiguousarray(_SEL.T)
_A24 = np.kron(np.eye(VC, dtype=np.float32), np.ones((1, 3), np.float32))
_A24T = np.ascontiguousarray(_A24.T)
_B3 = np.kron(np.ones((1, VC), np.float32), np.eye(3, dtype=np.float32))
_B16 = np.zeros((16, 24), np.float32)
_B16[6:9] = _B3
_B3_16 = np.zeros((16, 24), np.float32)
_B3_16[0:3] = _B3


def _lnk(x):
    m = jnp.mean(x, axis=-1, keepdims=True)
    xc = x - m
    v = jnp.mean(xc * xc, axis=-1, keepdims=True)
    return xc / jnp.sqrt(v + 1e-5)


def _mlp2k(x, w1, b1, w2, b2):
    h = jax.nn.relu(_lnk(x @ w1 + b1))
    return h @ w2 + b2


def _sc_mesh():
    return plsc.VectorSubcoreMesh(core_axis_name="c", subcore_axis_name="s")


@functools.partial(
    pl.kernel,
    mesh=_sc_mesh(),
    out_type=(jax.ShapeDtypeStruct((E, D), jnp.float32),
              jax.ShapeDtypeStruct((E, D), jnp.float32)),
    scratch_types=(
        [pltpu.VMEM((_IPW, _CH), jnp.int32),
         pltpu.VMEM((_IPW, _CH), jnp.int32)]
        + [pltpu.VMEM((_CH, D), jnp.float32)] * (2 * _K)
        + [pltpu.SemaphoreType.DMA] * (4 * _K)
    ),
)
def _sc_gather_pair(t1, i1, t2, i2, o1, o2, iv1, iv2, *bufs):
    r1 = bufs[0:_K]
    r2 = bufs[_K:2 * _K]
    s1 = bufs[2 * _K:3 * _K]
    s2 = bufs[3 * _K:4 * _K]
    w1 = bufs[4 * _K:5 * _K]
    w2 = bufs[5 * _K:6 * _K]
    wid = lax.axis_index("s") * _NC + lax.axis_index("c")
    base = wid * _PER
    pltpu.sync_copy(i1.at[wid], iv1)
    pltpu.sync_copy(i2.at[wid], iv2)

    def body(k, carry):
        hs = []
        for b in range(_K):
            j = k * _K + b
            hs.append((pltpu.async_copy(t1.at[iv1.at[j]], r1[b], s1[b]),
                       pltpu.async_copy(t2.at[iv2.at[j]], r2[b], s2[b])))
        ws = []
        for b in range(_K):
            j = k * _K + b
            h1, h2 = hs[b]
            h1.wait()
            ws.append(pltpu.async_copy(
                r1[b], o1.at[pl.ds(base + j * _CH, _CH), :], w1[b]))
            h2.wait()
            ws.append(pltpu.async_copy(
                r2[b], o2.at[pl.ds(base + j * _CH, _CH), :], w2[b]))
        for h in ws:
            h.wait()
        return carry

    lax.fori_loop(0, _NCHUNK // _K, body, 0)
    for t in range(_NCHUNK % _K):
        jt = (_NCHUNK // _K) * _K + t
        h1 = pltpu.async_copy(t1.at[iv1.at[jt]], r1[0], s1[0])
        h2 = pltpu.async_copy(t2.at[iv2.at[jt]], r2[0], s2[0])
        h1.wait()
        pltpu.sync_copy(r1[0], o1.at[pl.ds(base + jt * _CH, _CH), :])
        h2.wait()
        pltpu.sync_copy(r2[0], o2.at[pl.ds(base + jt * _CH, _CH), :])


@functools.partial(
    pl.kernel,
    mesh=_sc_mesh(),
    out_type=jax.ShapeDtypeStruct((_NC, N, _PC), jnp.float32),
    scratch_types=(
        [pltpu.VMEM((1, _CH), jnp.int32)] * _KS
        + [pltpu.VMEM((_CH, _PC), jnp.float32)] * _KS
        + [pltpu.VMEM_SHARED((_NACC, _PC), jnp.float32)]
        + [pltpu.SemaphoreType.DMA] * (3 * _KS)
    ),
)
def _sc_scatter_add(pay1, pay2, idx, zer, out, *bufs):
    ivr = bufs[0:_KS]
    pv = bufs[_KS:2 * _KS]
    acc = bufs[2 * _KS]
    si = bufs[2 * _KS + 1:3 * _KS + 1]
    sp = bufs[3 * _KS + 1:4 * _KS + 1]
    sa = bufs[4 * _KS + 1:5 * _KS + 1]
    cid = lax.axis_index("c")
    sid = lax.axis_index("s")

    pltpu.sync_copy(zer, acc.at[pl.ds(sid * _RPT, _RPT), :])
    plsc.subcore_barrier()

    def scatter_all(pay):
        def wbody(w, carry):
            base = w * _PER

            def chunk_group(k, nb):
                hs = []
                for b in range(nb):
                    j = k * _KS + b
                    hs.append(
                        (pltpu.async_copy(idx.at[w, j], ivr[b], si[b]),
                         pltpu.async_copy(
                             pay.at[pl.ds(base + j * _CH, _CH), :],
                             pv[b], sp[b])))
                has = []
                for b in range(nb):
                    hi, hp = hs[b]
                    hi.wait()
                    hp.wait()
                    has.append(pltpu.async_copy(
                        pv[b], acc.at[ivr[b].at[0]], sa[b], add=True))
                for h in has:
                    h.wait()

            def body(k, carry2):
                chunk_group(k, _KS)
                return carry2

            lax.fori_loop(0, _NCHUNK // _KS, body, 0)
            for t in range(_NCHUNK % _KS):
                jt = (_NCHUNK // _KS) * _KS + t
                hi = pltpu.async_copy(idx.at[w, jt], ivr[0], si[0])
                hp = pltpu.async_copy(
                    pay.at[pl.ds(base + jt * _CH, _CH), :], pv[0], sp[0])
                hi.wait()
                hp.wait()
                pltpu.sync_copy(pv[0], acc.at[ivr[0].at[0]], add=True)
            return carry

        lax.fori_loop(0, _NW, wbody, 0)

    @pl.when(jnp.logical_and(sid == 0, cid == 0))
    def _scatter_pay1():
        scatter_all(pay1)

    @pl.when(jnp.logical_and(sid == 0, cid == 1))
    def _scatter_pay2():
        scatter_all(pay2)

    plsc.subcore_barrier()

    @pl.when(sid < _NS - 1)
    def _dump_full():
        pltpu.sync_copy(acc.at[pl.ds(sid * _RPT, _RPT), :],
                        out.at[cid, pl.ds(sid * _RPT, _RPT), :])

    @pl.when(sid == _NS - 1)
    def _dump_tail():
        pltpu.sync_copy(acc.at[pl.ds(sid * _RPT, _DPT), :],
                        out.at[cid, pl.ds(sid * _RPT, _DPT), :])


def _embed_kernel(f_ref, x_ref, w8, br, b316, ver, qw1, qb1, qw2, qb2,
                  s_out, v_out, q_out):
    f = f_ref[...]
    s0 = f @ w8[...] + br[...]
    v0 = (x_ref[...] @ b316[...]) * ver[...]
    q0 = _mlp2k(s0, qw1[...], qb1[...], qw2[...], qb2[...])
    s_out[...] = s0
    v_out[...] = v0
    q_out[...] = q0


def _aux_kernel(xs_ref, xd_ref, ea_ref, aux_out):
    xs = xs_ref[...]
    xd = xd_ref[...]
    rel = xs[:, 0:3] - xd[:, 0:3]
    d2 = jnp.sum(rel * rel, axis=-1, keepdims=True) + 1e-12
    dist = jnp.sqrt(d2)
    ru = rel / (dist + 1e-8)
    ea = ea_ref[...][:, 0:5]
    zeros = jnp.zeros((ea.shape[0], 7), jnp.float32)
    aux_out[...] = jnp.concatenate([ea, dist, ru, zeros], axis=-1)


def _edge_kernel(s_ref, q_ref, aux_ref,
                 wk1a, wk1b, kb1, kw2, kb2,
                 wv1a, wv1b, vb1, vw2, vb2,
                 gw, gb, sel, selt, a24, b16, out_ref, out2_ref):
    s = s_ref[...]
    qd = q_ref[...]
    aux = aux_ref[...]
    hk = jax.nn.relu(_lnk(s @ wk1a[...] + aux @ wk1b[...] + kb1[...]))
    k = hk @ kw2[...] + kb2[...]
    hv = jax.nn.relu(_lnk(s @ wv1a[...] + aux @ wv1b[...] + vb1[...]))
    v = hv @ vw2[...] + vb2[...]
    logits = ((qd * k) @ sel[...]) * _INV_SQRT_DH
    ex = jnp.exp(logits)
    wv = v * (ex @ selt[...])
    gate = jnp.tanh(v @ gw[...] + gb[...])
    vm = (gate @ a24[...]) * (aux @ b16[...])
    zeros = jnp.zeros((s.shape[0], 100), jnp.float32)
    out_ref[...] = wv
    out2_ref[...] = jnp.concatenate([ex, vm, zeros], axis=-1)


def _node_kernel(part_ref, s_ref, v_ref,
                 selt, ow1, ob1, ow2, ob2, vsr, a24, a24t,
                 qw1, qb1, qw2, qb2,
                 s_out, v_out, q_out):
    pr = part_ref[...]
    numer = pr[0]
    sall2 = pr[1]
    den4 = sall2[:, 0:4]
    vagg = sall2[:, 4:28]
    agg = numer / (den4 @ selt[...] + 1e-9)
    upd = _mlp2k(agg, ow1[...], ob1[...], ow2[...], ob2[...])
    s1 = _lnk(s_ref[...] + upd)
    vr = v_ref[...] + vagg
    v2 = (vr * vr) @ a24t[...]
    vnorm = jnp.sqrt(v2 + 1e-12)
    fac = vsr[...] / (1.0 + vnorm)
    v1 = vr * (fac @ a24[...])
    qn = _mlp2k(s1, qw1[...], qb1[...], qw2[...], qb2[...])
    s_out[...] = s1
    v_out[...] = v1
    q_out[...] = qn


def _feat_kernel(s_ref, v_ref, a24t, w1a, w1b, b1, w2, b2, f_out):
    sc = s_ref[...]
    vr = v_ref[...]
    vinv = jnp.sqrt((vr * vr) @ a24t[...] + 1e-12)
    h = jax.nn.relu(_lnk(sc @ w1a[...] + vinv @ w1b[...] + b1[...]))
    f_out[...] = h @ w2[...] + b2[...] + sc


def _pool_kernel(ids_ref, f_ref, gw1, gb1, gw2, gb2, out_ref):
    step = pl.program_id(0)

    @pl.when(step == 0)
    def _init():
        out_ref[...] = jnp.full((G, D), -1e30, jnp.float32)

    def body(r, carry):
        g = ids_ref[0, 0, r]
        row = f_ref[pl.ds(r, 1), :]
        out_ref[pl.ds(g, 1), :] = jnp.maximum(out_ref[pl.ds(g, 1), :], row)
        return carry

    lax.fori_loop(0, _BN, body, 0)

    @pl.when(step == pl.num_programs(0) - 1)
    def _head():
        pooled = out_ref[...]
        pooled = jnp.where(pooled == -1e30, 0.0, pooled)
        h = jax.nn.relu(_lnk(pooled @ gw1[...] + gb1[...]))
        out_ref[...] = h @ gw2[...] + gb2[...]


def _full(shape):
    nd = len(shape)
    return pl.BlockSpec(shape, lambda i, _nd=nd: (0,) * _nd)


def _rows(block, ncols):
    return pl.BlockSpec((block, ncols), lambda i: (i, 0))


def _tc_embed(f8, x16, w8, br, b316, ver, qw1, qb1, qw2, qb2):
    return pl.pallas_call(
        _embed_kernel,
        grid=(N // _BN,),
        in_specs=[_rows(_BN, 8), _rows(_BN, 16),
                  _full((8, D)), _full((1, D)), _full((16, 24)), _full((1, 24)),
                  _full((D, D)), _full((1, D)), _full((D, D)), _full((1, D))],
        out_specs=[_rows(_BN, D), _rows(_BN, 24), _rows(_BN, D)],
        out_shape=[jax.ShapeDtypeStruct((N, D), jnp.float32),
                   jax.ShapeDtypeStruct((N, 24), jnp.float32),
                   jax.ShapeDtypeStruct((N, D), jnp.float32)],
    )(f8, x16, w8, br, b316, ver, qw1, qb1, qw2, qb2)


def _tc_aux(xs16, xd16, ea8):
    return pl.pallas_call(
        _aux_kernel,
        grid=(E // _BE,),
        in_specs=[_rows(_BE, D), _rows(_BE, D), _rows(_BE, 8)],
        out_specs=_rows(_BE, 16),
        out_shape=jax.ShapeDtypeStruct((E, 16), jnp.float32),
    )(xs16, xd16, ea8)


def _tc_edge(ssrc, qdst, aux, wk1a, wk1b, kb1, kw2, kb2,
             wv1a, wv1b, vb1, vw2, vb2, gw, gb, sel, selt, a24, b16):
    return pl.pallas_call(
        _edge_kernel,
        grid=(E // _BE,),
        in_specs=[_rows(_BE, D), _rows(_BE, D), _rows(_BE, 16),
                  _full((D, D)), _full((16, D)), _full((1, D)),
                  _full((D, D)), _full((1, D)),
                  _full((D, D)), _full((16, D)), _full((1, D)),
                  _full((D, D)), _full((1, D)),
                  _full((D, VC)), _full((1, VC)),
                  _full((D, H)), _full((H, D)),
                  _full((VC, 24)), _full((16, 24))],
        out_specs=[_rows(_BE, _PC), _rows(_BE, _PC)],
        out_shape=[jax.ShapeDtypeStruct((E, _PC), jnp.float32),
                   jax.ShapeDtypeStruct((E, _PC), jnp.float32)],
    )(ssrc, qdst, aux, wk1a, wk1b, kb1, kw2, kb2,
      wv1a, wv1b, vb1, vw2, vb2, gw, gb, sel, selt, a24, b16)


def _tc_node(part, scalar, vec, selt, ow1, ob1, ow2, ob2, vsr, a24, a24t,
             qw1, qb1, qw2, qb2):
    return pl.pallas_call(
        _node_kernel,
        grid=(N // _BN,),
        in_specs=[pl.BlockSpec((_NC, _BN, _PC), lambda i: (0, i, 0)),
                  _rows(_BN, D), _rows(_BN, 24),
                  _full((H, D)),
                  _full((D, D)), _full((1, D)), _full((D, D)), _full((1, D)),
                  _full((1, VC)), _full((VC, 24)), _full((24, VC)),
                  _full((D, D)), _full((1, D)), _full((D, D)), _full((1, D))],
        out_specs=[_rows(_BN, D), _rows(_BN, 24), _rows(_BN, D)],
        out_shape=[jax.ShapeDtypeStruct((N, D), jnp.float32),
                   jax.ShapeDtypeStruct((N, 24), jnp.float32),
                   jax.ShapeDtypeStruct((N, D), jnp.float32)],
    )(part, scalar, vec, selt, ow1, ob1, ow2, ob2, vsr, a24, a24t,
      qw1, qb1, qw2, qb2)


def _tc_feat(scalar, vec, a24t, w1a, w1b, b1, w2, b2):
    return pl.pallas_call(
        _feat_kernel,
        grid=(N // _BN,),
        in_specs=[_rows(_BN, D), _rows(_BN, 24), _full((24, VC)),
                  _full((D, D)), _full((VC, D)), _full((1, D)),
                  _full((D, D)), _full((1, D))],
        out_specs=_rows(_BN, D),
        out_shape=jax.ShapeDtypeStruct((N, D), jnp.float32),
    )(scalar, vec, a24t, w1a, w1b, b1, w2, b2)


def _tc_pool(ids3, feat, gw1, gb1, gw2, gb2):
    return pl.pallas_call(
        _pool_kernel,
        grid=(N // _BN,),
        in_specs=[pl.BlockSpec((1, 1, _BN), lambda i: (i, 0, 0),
                               memory_space=pltpu.SMEM),
                  _rows(_BN, D),
                  _full((D, D)), _full((1, D)), _full((D, D)), _full((1, D))],
        out_specs=_full((G, D)),
        out_shape=jax.ShapeDtypeStruct((G, D), jnp.float32),
    )(ids3, feat, gw1, gb1, gw2, gb2)


def kernel(node_f, node_x, edge_index, edge_attr, graph_ids, params):
    p = params
    src2 = edge_index[0].reshape(_NW, _IPW, _CH).astype(jnp.int32)
    dst2 = edge_index[1].reshape(_NW, _IPW, _CH).astype(jnp.int32)
    dst4 = dst2.reshape(_NW, _IPW, 1, _CH)

    f8 = jnp.pad(node_f[:, :, 0], ((0, 0), (0, 2)))
    x16 = jnp.pad(node_x, ((0, 0), (0, 13)))
    ea8 = jnp.pad(edge_attr, ((0, 0), (0, 3)))
    zer = jnp.zeros((_RPT, _PC), jnp.float32)

    sel = jnp.asarray(_SEL)
    selt = jnp.asarray(_SELT)
    a24 = jnp.asarray(_A24)
    a24t = jnp.asarray(_A24T)
    b16 = jnp.asarray(_B16)
    b316 = jnp.asarray(_B3_16)

    embw = p['embed_W']
    embw = embw.at[5].set(embw[5] / 9.0)
    w8 = jnp.concatenate([embw, jnp.zeros((2, D), jnp.float32)], axis=0)
    br = p['embed_b'][None, :]
    ver = jnp.repeat(p['vec_embed'][0], 3)[None, :]

    lps = [p['l%d' % l] for l in range(4)]
    lp0 = lps[0]
    scalar, vec, q = _tc_embed(
        f8, x16, w8, br, b316, ver,
        lp0['qW1'], lp0['qb1'][None, :], lp0['qW2'], lp0['qb2'][None, :])

    x128 = jnp.pad(node_x, ((0, 0), (0, D - 3)))
    xs, xd = _sc_gather_pair(x128, src2, x128, dst2)
    aux = _tc_aux(xs, xd, ea8)

    for l in range(4):
        lp = lps[l]
        wk1b = jnp.concatenate(
            [lp['kW1'][D:D + 6], jnp.zeros((10, D), jnp.float32)], axis=0)
        wv1b = jnp.concatenate(
            [lp['vW1'][D:D + 6], jnp.zeros((10, D), jnp.float32)], axis=0)

        ssrc, qdst = _sc_gather_pair(scalar, src2, q, dst2)
        pay, pay2 = _tc_edge(
            ssrc, qdst, aux,
            lp['kW1'][:D], wk1b, lp['kb1'][None, :], lp['kW2'], lp['kb2'][None, :],
            lp['vW1'][:D], wv1b, lp['vb1'][None, :], lp['vW2'], lp['vb2'][None, :],
            lp['gW'], lp['gb'][None, :], sel, selt, a24, b16)
        part = _sc_scatter_add(pay, pay2, dst4, zer)

        lpn = lps[(l + 1) % 4]
        scalar, vec, q = _tc_node(
            part, scalar, vec, selt,
            lp['oW1'], lp['ob1'][None, :], lp['oW2'], lp['ob2'][None, :],
            lp['vscale'][None, :], a24, a24t,
            lpn['qW1'], lpn['qb1'][None, :], lpn['qW2'], lpn['qb2'][None, :])

    nmw1b = p['nmW1'][D:D + VC]
    feat = _tc_feat(scalar, vec, a24t,
                    p['nmW1'][:D], nmw1b, p['nmb1'][None, :],
                    p['nmW2'], p['nmb2'][None, :])

    gw2 = jnp.zeros((D, D), jnp.float32).at[:, 0:1].set(p['gmW2'])
    gb2 = jnp.zeros((1, D), jnp.float32).at[0, 0].set(p['gmb2'][0])
    ids3 = graph_ids.reshape(N // _BN, 1, _BN).astype(jnp.int32)
    out = _tc_pool(ids3, feat, p['gmW1'], p['gmb1'][None, :], gw2, gb2)
    return out[:, 0:1]
